# baseline (speedup 1.0000x reference)
import functools

import jax
import jax.numpy as jnp
from jax import lax
from jax.experimental import pallas as pl
from jax.experimental.pallas import tpu as pltpu
from jax.experimental.pallas import tpu_sc as plsc

EMBED_DIM = 64
NUM_CORES = 2
NUM_SUBCORES = 16
NUM_WORKERS = NUM_CORES * NUM_SUBCORES
CHUNK = 128


@functools.partial(jax.jit, static_argnums=(2, 3))
def _sc_gather(idx, table, nchunk, per_w):
  n = NUM_WORKERS * per_w
  mesh = plsc.VectorSubcoreMesh(
      core_axis_name="c", subcore_axis_name="s",
      num_cores=NUM_CORES, num_subcores=NUM_SUBCORES)

  @functools.partial(
      pl.kernel,
      out_type=jax.ShapeDtypeStruct((n, EMBED_DIM), jnp.float32),
      mesh=mesh,
      compiler_params=pltpu.CompilerParams(use_tc_tiling_on_sc=False),
      scratch_types=[
          pltpu.VMEM((nchunk, CHUNK), jnp.int32),
          pltpu.VMEM((CHUNK, EMBED_DIM), jnp.float32),
          pltpu.SemaphoreType.DMA,
      ],
  )
  def k(idx_hbm, table_hbm, out_hbm, idx_v, rows_v, gsem):
    wid = lax.axis_index("s") * NUM_CORES + lax.axis_index("c")
    base = wid * per_w
    pltpu.sync_copy(idx_hbm.at[wid], idx_v)

    def body(j, _):
      pltpu.async_copy(table_hbm.at[idx_v.at[j]], rows_v, gsem).wait()
      pltpu.sync_copy(rows_v, out_hbm.at[pl.ds(base + j * CHUNK, CHUNK)])
      return 0

    lax.fori_loop(0, nchunk, body, 0)

  return k(idx, table)


def kernel(indices, feature_tensor):
  b, h = indices.shape
  n = b * h
  per_w = n // NUM_WORKERS
  nchunk = per_w // CHUNK
  idx = indices.astype(jnp.int32).reshape(NUM_WORKERS, nchunk, CHUNK)
  out = _sc_gather(idx, feature_tensor, nchunk, per_w)
  return out.reshape(b, h, EMBED_DIM)


# speedup vs baseline: 1.1133x; 1.1133x over previous
import functools

import jax
import jax.numpy as jnp
from jax import lax
from jax.experimental import pallas as pl
from jax.experimental.pallas import tpu as pltpu
from jax.experimental.pallas import tpu_sc as plsc

EMBED_DIM = 64
NUM_CORES = 2
NUM_SUBCORES = 16
NUM_WORKERS = NUM_CORES * NUM_SUBCORES
CHUNK = 128
NBUF = 8
LOOKAHEAD = 6


@functools.partial(jax.jit, static_argnums=(2, 3))
def _sc_gather(idx, table, nchunk, per_w):
  n = NUM_WORKERS * per_w
  mesh = plsc.VectorSubcoreMesh(
      core_axis_name="c", subcore_axis_name="s",
      num_cores=NUM_CORES, num_subcores=NUM_SUBCORES)

  @functools.partial(
      pl.kernel,
      out_type=jax.ShapeDtypeStruct((n, EMBED_DIM), jnp.float32),
      mesh=mesh,
      compiler_params=pltpu.CompilerParams(use_tc_tiling_on_sc=False),
      scratch_types=[
          pltpu.VMEM((nchunk, CHUNK), jnp.int32),
          pltpu.VMEM((NBUF, CHUNK, EMBED_DIM), jnp.float32),
          pltpu.SemaphoreType.DMA((NBUF,)),
          pltpu.SemaphoreType.DMA((NBUF,)),
      ],
  )
  def k(idx_hbm, table_hbm, out_hbm, idx_v, rows_v, gsem, ssem):
    wid = lax.axis_index("s") * NUM_CORES + lax.axis_index("c")
    base = wid * per_w
    pltpu.sync_copy(idx_hbm.at[wid], idx_v)

    def fire_gather(j, slot):
      pltpu.async_copy(table_hbm.at[idx_v.at[j]], rows_v.at[slot],
                       gsem.at[slot])

    for b in range(LOOKAHEAD):
      fire_gather(b, b)

    ngroups = nchunk // NBUF

    def group(g, _):
      j0 = g * NBUF
      for b in range(NBUF):
        j = j0 + b
        pltpu.make_async_copy(table_hbm.at[idx_v.at[j]], rows_v.at[b],
                              gsem.at[b]).wait()
        pltpu.async_copy(rows_v.at[b], out_hbm.at[pl.ds(base + j * CHUNK,
                                                        CHUNK)],
                         ssem.at[b])
        nslot = (b + LOOKAHEAD) % NBUF
        jn = j + LOOKAHEAD
        jprev = jn - NBUF

        @pl.when(jn < nchunk)
        def _():
          @pl.when(jprev >= 0)
          def _():
            pltpu.make_async_copy(
                rows_v.at[nslot],
                out_hbm.at[pl.ds(base + jprev * CHUNK, CHUNK)],
                ssem.at[nslot]).wait()
          fire_gather(jn, nslot)
      return 0

    lax.fori_loop(0, ngroups, group, 0)

    tail = nchunk % NBUF
    j0 = ngroups * NBUF
    for t in range(tail):
      j = j0 + t
      b = j % NBUF
      pltpu.make_async_copy(table_hbm.at[idx_v.at[j]], rows_v.at[b],
                            gsem.at[b]).wait()
      pltpu.async_copy(rows_v.at[b], out_hbm.at[pl.ds(base + j * CHUNK,
                                                      CHUNK)],
                       ssem.at[b])
      nslot = (b + LOOKAHEAD) % NBUF
      jn = j + LOOKAHEAD
      if jn < nchunk:
        jprev = jn - NBUF
        if jprev >= 0:
          pltpu.make_async_copy(
              rows_v.at[nslot],
              out_hbm.at[pl.ds(base + jprev * CHUNK, CHUNK)],
              ssem.at[nslot]).wait()
        fire_gather(jn, nslot)
    for d in range(min(NBUF, nchunk)):
      j = nchunk - 1 - d
      b = j % NBUF
      pltpu.make_async_copy(rows_v.at[b],
                            out_hbm.at[pl.ds(base + j * CHUNK, CHUNK)],
                            ssem.at[b]).wait()

  return k(idx, table)


def kernel(indices, feature_tensor):
  b, h = indices.shape
  n = b * h
  per_w = n // NUM_WORKERS
  nchunk = per_w // CHUNK
  idx = indices.astype(jnp.int32).reshape(NUM_WORKERS, nchunk, CHUNK)
  out = _sc_gather(idx, feature_tensor, nchunk, per_w)
  return out.reshape(b, h, EMBED_DIM)


# speedup vs baseline: 1.1493x; 1.0324x over previous
import functools

import jax
import jax.numpy as jnp
from jax import lax
from jax.experimental import pallas as pl
from jax.experimental.pallas import tpu as pltpu
from jax.experimental.pallas import tpu_sc as plsc

EMBED_DIM = 64
NUM_CORES = 2
NUM_SUBCORES = 16
NUM_WORKERS = NUM_CORES * NUM_SUBCORES
CW = 128
NBUF = 8
LOOKAHEAD = 6


@functools.partial(jax.jit, static_argnums=(2, 3))
def _sc_gather(idx_flat, table, bsz, hlen):
  per_w = (bsz // NUM_WORKERS) * hlen
  nchunk = per_w // CW
  mesh = plsc.VectorSubcoreMesh(
      core_axis_name="c", subcore_axis_name="s",
      num_cores=NUM_CORES, num_subcores=NUM_SUBCORES)

  @functools.partial(
      pl.kernel,
      out_type=jax.ShapeDtypeStruct((hlen, bsz, EMBED_DIM), jnp.float32),
      mesh=mesh,
      compiler_params=pltpu.CompilerParams(use_tc_tiling_on_sc=False),
      scratch_types=[
          pltpu.VMEM((per_w,), jnp.int32),
          pltpu.VMEM((NBUF, CW, EMBED_DIM), jnp.float32),
          pltpu.SemaphoreType.DMA((NBUF,)),
          pltpu.SemaphoreType.DMA((NBUF,)),
      ],
  )
  def k(idx_hbm, table_hbm, out_hbm, idx_v, rows_v, gsem, ssem):
    wid = lax.axis_index("s") * NUM_CORES + lax.axis_index("c")
    col0 = wid * CW
    pltpu.sync_copy(idx_hbm.at[pl.ds(wid * per_w, per_w)], idx_v)

    def out_slab(h):
      return out_hbm.at[h, pl.ds(col0, CW)]

    def fire_gather(j, slot):
      off = pl.multiple_of(j * CW, CW)
      pltpu.async_copy(table_hbm.at[idx_v.at[pl.ds(off, CW)]],
                       rows_v.at[slot], gsem.at[slot])

    for b in range(LOOKAHEAD):
      fire_gather(jnp.int32(b), b)

    ngroups = nchunk // NBUF

    def group(g, _):
      j0 = g * NBUF
      for u in range(NBUF):
        j = j0 + u
        pltpu.make_async_copy(table_hbm.at[idx_v.at[pl.ds(0, CW)]],
                              rows_v.at[u], gsem.at[u]).wait()
        pltpu.async_copy(rows_v.at[u], out_slab(j), ssem.at[u])
        nslot = (u + LOOKAHEAD) % NBUF
        jn = j + LOOKAHEAD
        jprev = jn - NBUF

        @pl.when(jn < nchunk)
        def _():
          @pl.when(jprev >= 0)
          def _():
            pltpu.make_async_copy(rows_v.at[nslot], out_slab(jprev),
                                  ssem.at[nslot]).wait()
          fire_gather(jn, nslot)
      return 0

    lax.fori_loop(0, ngroups, group, 0)

    for d in range(NBUF):
      j = nchunk - NBUF + d
      pltpu.make_async_copy(rows_v.at[j % NBUF], out_slab(jnp.int32(j)),
                            ssem.at[j % NBUF]).wait()

  return k(idx_flat, table)


def kernel(indices, feature_tensor):
  bsz, hlen = indices.shape
  v = feature_tensor.shape[0]
  idx_t = jnp.swapaxes(indices.astype(jnp.int32), 0, 1)
  idx_flat = (idx_t.reshape(hlen, NUM_WORKERS, CW)
              .transpose(1, 0, 2).reshape(-1))
  del v
  out3 = _sc_gather(idx_flat, feature_tensor, bsz, hlen)
  return jnp.transpose(out3, (1, 0, 2))
